# Initial kernel scaffold; baseline (speedup 1.0000x reference)
import jax
import jax.numpy as jnp
from jax.experimental import pallas as pl


def kernel(x, edge_index, edge_attr, batch, Wa, ba, W1, b1, W2, b2, We1, be1, We2, be2, We3, be3, Wh1, bh1, Wh2, bh2):
    raise NotImplementedError("write your pallas kernel here")


# speedup vs baseline: 2.6858x; 2.6858x over previous
import functools

import jax
import jax.numpy as jnp
from jax import lax
from jax.experimental import pallas as pl
from jax.experimental.pallas import tpu as pltpu
from jax.experimental.pallas import tpu_sc as plsc

N = 10000
E = 320000
D = 128
ED = 4
G = 64

C = 128
NCHUNK = E // C
NW = 32
CPW = -(-NCHUNK // NW)
NSUB = 16
RPT = 624
RREM = N - NSUB * RPT


def _make_sc_layer():
    mesh = plsc.VectorSubcoreMesh(core_axis_name="c", subcore_axis_name="s")

    @functools.partial(
        pl.kernel,
        mesh=mesh,
        out_type=jax.ShapeDtypeStruct((2, N, D), jnp.float32),
        scratch_types=[
            pltpu.VMEM((C,), jnp.int32),
            pltpu.VMEM((C,), jnp.int32),
            pltpu.VMEM((C * ED + 16,), jnp.float32),
            pltpu.VMEM((C, D), jnp.float32),
            pltpu.VMEM((ED * D,), jnp.float32),
            pltpu.VMEM((D,), jnp.float32),
            pltpu.VMEM_SHARED((N, D), jnp.float32),
            pltpu.SemaphoreType.DMA,
        ],
    )
    def sc_layer(h_hbm, si_hbm, di_hbm, ea_hbm, we_hbm, be_hbm, z_hbm, out_hbm,
                 src_v, dst_v, attr_v, rows_v, we_v, be_v, acc_sh, sem):
        cid = lax.axis_index("c")
        sid = lax.axis_index("s")
        wid = sid * 2 + cid

        r0 = sid * RPT
        pltpu.sync_copy(z_hbm.at[pl.ds(r0, RPT)], acc_sh.at[pl.ds(r0, RPT)])

        @pl.when(sid == 0)
        def _():
            pltpu.sync_copy(z_hbm.at[pl.ds(NSUB * RPT, RREM)],
                            acc_sh.at[pl.ds(NSUB * RPT, RREM)])

        pltpu.sync_copy(we_hbm, we_v)
        pltpu.sync_copy(be_hbm, be_v)
        plsc.subcore_barrier()

        wv = [[we_v[pl.ds(k * D + g * 16, 16)] for g in range(8)]
              for k in range(ED)]
        bv = [be_v[pl.ds(g * 16, 16)] for g in range(8)]

        def chunk_body(i, carry):
            c = wid + NW * i

            @pl.when(c < NCHUNK)
            def _():
                base = c * C
                pltpu.sync_copy(si_hbm.at[pl.ds(base, C)], src_v)
                pltpu.sync_copy(di_hbm.at[pl.ds(base, C)], dst_v)
                pltpu.sync_copy(ea_hbm.at[pl.ds(base * ED, C * ED)],
                                attr_v.at[pl.ds(0, C * ED)])
                pltpu.async_copy(h_hbm.at[src_v], rows_v, sem).wait()

                def edge_body(j, ecarry):
                    av = attr_v[pl.ds(j * ED, 16)]
                    a = [av[k] for k in range(ED)]
                    for g in range(8):
                        hv = rows_v[j, pl.ds(g * 16, 16)]
                        e = (bv[g] + a[0] * wv[0][g] + a[1] * wv[1][g]
                             + a[2] * wv[2][g] + a[3] * wv[3][g])
                        rows_v[j, pl.ds(g * 16, 16)] = jnp.maximum(hv + e, 0.0)
                    return ecarry

                lax.fori_loop(0, C, edge_body, 0, unroll=False)
                pltpu.sync_copy(rows_v, acc_sh.at[dst_v], add=True)

            return carry

        lax.fori_loop(0, CPW, chunk_body, 0, unroll=False)

        plsc.subcore_barrier()
        pltpu.sync_copy(acc_sh.at[pl.ds(r0, RPT)],
                        out_hbm.at[cid, pl.ds(r0, RPT)])

        @pl.when(sid == 0)
        def _():
            pltpu.sync_copy(acc_sh.at[pl.ds(NSUB * RPT, RREM)],
                            out_hbm.at[cid, pl.ds(NSUB * RPT, RREM)])

    return sc_layer


_sc_layer = _make_sc_layer()


def _proj_body(x_ref, w_ref, b_ref, o_ref):
    acc = lax.dot_general(x_ref[...], w_ref[...], (((1,), (0,)), ((), ())),
                          preferred_element_type=jnp.float32)
    o_ref[...] = jnp.maximum(acc + b_ref[...], 0.0)


def _proj(xp, wap, ba2):
    return pl.pallas_call(
        _proj_body,
        out_shape=jax.ShapeDtypeStruct((N, D), jnp.float32),
    )(xp, wap, ba2)


_BLK = 2000


def _mlp_body(h_ref, agg_ref, w1_ref, b1_ref, w2_ref, b2_ref, o_ref):
    z = h_ref[...] + agg_ref[0] + agg_ref[1]
    z = jnp.maximum(
        lax.dot_general(z, w1_ref[...], (((1,), (0,)), ((), ())),
                        preferred_element_type=jnp.float32) + b1_ref[...], 0.0)
    z = lax.dot_general(z, w2_ref[...], (((1,), (0,)), ((), ())),
                        preferred_element_type=jnp.float32) + b2_ref[...]
    o_ref[...] = jnp.maximum(z, 0.0)


def _mlp(h, agg, w1, b12, w2, b22):
    return pl.pallas_call(
        _mlp_body,
        grid=(N // _BLK,),
        in_specs=[
            pl.BlockSpec((_BLK, D), lambda i: (i, 0)),
            pl.BlockSpec((2, _BLK, D), lambda i: (0, i, 0)),
            pl.BlockSpec((D, D), lambda i: (0, 0)),
            pl.BlockSpec((1, D), lambda i: (0, 0)),
            pl.BlockSpec((D, D), lambda i: (0, 0)),
            pl.BlockSpec((1, D), lambda i: (0, 0)),
        ],
        out_specs=pl.BlockSpec((_BLK, D), lambda i: (i, 0)),
        out_shape=jax.ShapeDtypeStruct((N, D), jnp.float32),
    )(h, agg, w1, b12, w2, b22)


def _pool_body(h_ref, b_ref, wh1_ref, bh1_ref, wh2_ref, bh2_ref, o_ref):
    bt = b_ref[...]
    io = lax.broadcasted_iota(jnp.int32, (G, N), 0)
    oht = (bt == io).astype(jnp.float32)
    g = lax.dot_general(oht, h_ref[...], (((1,), (0,)), ((), ())),
                        preferred_element_type=jnp.float32)
    q = jnp.maximum(
        lax.dot_general(g, wh1_ref[...], (((1,), (0,)), ((), ())),
                        preferred_element_type=jnp.float32) + bh1_ref[...], 0.0)
    o_ref[...] = lax.dot_general(q, wh2_ref[...], (((1,), (0,)), ((), ())),
                                 preferred_element_type=jnp.float32) + bh2_ref[...]


def _pool(h, batch2, wh1, bh12, wh2, bh22):
    return pl.pallas_call(
        _pool_body,
        out_shape=jax.ShapeDtypeStruct((G, 1), jnp.float32),
    )(h, batch2, wh1, bh12, wh2, bh22)


def kernel(x, edge_index, edge_attr, batch, Wa, ba, W1, b1, W2, b2,
           We1, be1, We2, be2, We3, be3, Wh1, bh1, Wh2, bh2):
    xp = jnp.pad(x, ((0, 0), (0, 16 - x.shape[1])))
    wap = jnp.pad(Wa, ((0, 16 - Wa.shape[0]), (0, 0)))
    h = _proj(xp, wap, ba.reshape(1, D))

    ea = edge_attr.reshape(-1)
    zeros = jnp.zeros((N, D), jnp.float32)
    b12 = b1.reshape(1, D)
    b22 = b2.reshape(1, D)
    si = edge_index[0]
    di = edge_index[1]
    for We, be in ((We1, be1), (We2, be2), (We3, be3)):
        agg = _sc_layer(h, si, di, ea, We.reshape(-1), be, zeros)
        h = _mlp(h, agg, W1, b12, W2, b22)

    out = _pool(h, batch.reshape(1, N), Wh1, bh1.reshape(1, G),
                Wh2, bh2.reshape(1, 1))
    return out.reshape(-1)


# speedup vs baseline: 3.4343x; 1.2787x over previous
import functools

import jax
import jax.numpy as jnp
from jax import lax
from jax.experimental import pallas as pl
from jax.experimental.pallas import tpu as pltpu
from jax.experimental.pallas import tpu_sc as plsc

N = 10000
E = 320000
D = 128
ED = 4
G = 64

C = 120
NW = 32
KPW = 84
NT = KPW // 3
EPAD = NW * KPW * C
NP = N + 8
NSUB = 16
RPT = 624
RREM = N - NSUB * RPT
ZREM = NP - NSUB * RPT
APAD = C * ED + 16


def _make_sc_layer():
    mesh = plsc.VectorSubcoreMesh(core_axis_name="c", subcore_axis_name="s")

    @functools.partial(
        pl.kernel,
        mesh=mesh,
        out_type=jax.ShapeDtypeStruct((2, N, D), jnp.float32),
        scratch_types=[
            pltpu.VMEM((3, C, D), jnp.float32),
            pltpu.VMEM((C,), jnp.int32),
            pltpu.VMEM((C,), jnp.int32),
            pltpu.VMEM((C,), jnp.int32),
            pltpu.VMEM((C,), jnp.int32),
            pltpu.VMEM((C,), jnp.int32),
            pltpu.VMEM((C,), jnp.int32),
            pltpu.VMEM((APAD,), jnp.float32),
            pltpu.VMEM((APAD,), jnp.float32),
            pltpu.VMEM((APAD,), jnp.float32),
            pltpu.VMEM((ED * D,), jnp.float32),
            pltpu.VMEM((D,), jnp.float32),
            pltpu.VMEM_SHARED((NP, D), jnp.float32),
            pltpu.SemaphoreType.DMA,
            pltpu.SemaphoreType.DMA,
            pltpu.SemaphoreType.DMA,
            pltpu.SemaphoreType.DMA,
            pltpu.SemaphoreType.DMA,
            pltpu.SemaphoreType.DMA,
            pltpu.SemaphoreType.DMA,
            pltpu.SemaphoreType.DMA,
            pltpu.SemaphoreType.DMA,
        ],
    )
    def sc_layer(h_hbm, si_hbm, di_hbm, ea_hbm, we_hbm, be_hbm, z_hbm, out_hbm,
                 rows3, src0, src1, src2, dst0, dst1, dst2, at0, at1, at2,
                 we_v, be_v, acc_sh,
                 sg0, sg1, sg2, ss0, ss1, ss2, si0, si1, si2):
        cid = lax.axis_index("c")
        sid = lax.axis_index("s")
        w = sid * 2 + cid
        rows = [rows3.at[0], rows3.at[1], rows3.at[2]]
        srcs = [src0, src1, src2]
        dsts = [dst0, dst1, dst2]
        ats = [at0, at1, at2]
        sgs = [sg0, sg1, sg2]
        sss = [ss0, ss1, ss2]
        sis = [si0, si1, si2]

        r0 = sid * RPT
        pltpu.sync_copy(z_hbm.at[pl.ds(r0, RPT)], acc_sh.at[pl.ds(r0, RPT)])

        @pl.when(sid == 0)
        def _():
            pltpu.sync_copy(z_hbm.at[pl.ds(NSUB * RPT, ZREM)],
                            acc_sh.at[pl.ds(NSUB * RPT, ZREM)])

        pltpu.sync_copy(we_hbm, we_v)
        pltpu.sync_copy(be_hbm, be_v)
        plsc.subcore_barrier()

        wv = [[we_v[pl.ds(k * D + g * 16, 16)] for g in range(8)]
              for k in range(ED)]
        bv = [be_v[pl.ds(g * 16, 16)] for g in range(8)]
        ebase = w * KPW * C

        def is_real(k):
            return ebase + k * C < E

        def issue_idx(k, p):
            base = ebase + k * C
            pltpu.async_copy(si_hbm.at[pl.ds(base, C)], srcs[p], sis[p])
            pltpu.async_copy(di_hbm.at[pl.ds(base, C)], dsts[p], sis[p])
            pltpu.async_copy(ea_hbm.at[pl.ds(base * ED, C * ED)],
                             ats[p].at[pl.ds(0, C * ED)], sis[p])

        def wait_idx(p):
            pltpu.make_async_copy(si_hbm.at[pl.ds(0, C)], srcs[p], sis[p]).wait()
            pltpu.make_async_copy(di_hbm.at[pl.ds(0, C)], dsts[p], sis[p]).wait()
            pltpu.make_async_copy(ea_hbm.at[pl.ds(0, C * ED)],
                                  ats[p].at[pl.ds(0, C * ED)], sis[p]).wait()

        def issue_gather(p):
            pltpu.async_copy(h_hbm.at[srcs[p]], rows[p], sgs[p])

        def wait_gather(p):
            pltpu.make_async_copy(h_hbm.at[pl.ds(0, C)], rows[p], sgs[p]).wait()

        def issue_scatter(p):
            pltpu.async_copy(rows[p], acc_sh.at[dsts[p]], sss[p], add=True)

        def wait_scatter(p):
            pltpu.make_async_copy(h_hbm.at[pl.ds(0, C)], rows[p], sss[p]).wait()

        def compute(p):
            rp = rows[p]
            ap = ats[p]

            def edge_body(j, carry):
                av = ap[pl.ds(j * ED, 16)]
                a = [av[t] for t in range(ED)]
                for g in range(8):
                    hv = rp[j, pl.ds(g * 16, 16)]
                    e = (bv[g] + a[0] * wv[0][g] + a[1] * wv[1][g]
                         + a[2] * wv[2][g] + a[3] * wv[3][g])
                    rp[j, pl.ds(g * 16, 16)] = jnp.maximum(hv + e, 0.0)
                return carry

            lax.fori_loop(0, C, edge_body, 0, unroll=2)

        for k in (0, 1, 2):
            @pl.when(is_real(k))
            def _(k=k):
                issue_idx(k, k)

        for k in (0, 1):
            @pl.when(is_real(k))
            def _(k=k):
                wait_idx(k)
                issue_gather(k)

        def triplet(kt, carry):
            k0 = 3 * kt
            k1 = k0 + 1
            k2 = k0 + 2

            @pl.when(is_real(k0))
            def _():
                wait_gather(0)

            @pl.when(is_real(k0 + 3) & (k0 + 3 < KPW))
            def _():
                issue_idx(k0 + 3, 0)

            @pl.when(is_real(k0))
            def _():
                compute(0)
                issue_scatter(0)

            @pl.when((kt > 0) & is_real(k2 - 3))
            def _():
                wait_scatter(2)

            @pl.when(is_real(k2))
            def _():
                wait_idx(2)
                issue_gather(2)

            @pl.when(is_real(k1))
            def _():
                wait_gather(1)

            @pl.when(is_real(k1 + 3) & (k1 + 3 < KPW))
            def _():
                issue_idx(k1 + 3, 1)

            @pl.when(is_real(k1))
            def _():
                compute(1)
                issue_scatter(1)

            @pl.when(is_real(k0))
            def _():
                wait_scatter(0)

            @pl.when(is_real(k0 + 3) & (k0 + 3 < KPW))
            def _():
                wait_idx(0)
                issue_gather(0)

            @pl.when(is_real(k2))
            def _():
                wait_gather(2)

            @pl.when(is_real(k2 + 3) & (k2 + 3 < KPW))
            def _():
                issue_idx(k2 + 3, 2)

            @pl.when(is_real(k2))
            def _():
                compute(2)
                issue_scatter(2)

            @pl.when(is_real(k1))
            def _():
                wait_scatter(1)

            @pl.when(is_real(k1 + 3) & (k1 + 3 < KPW))
            def _():
                wait_idx(1)
                issue_gather(1)

            return carry

        lax.fori_loop(0, NT, triplet, 0, unroll=False)

        @pl.when(is_real(KPW - 1))
        def _():
            wait_scatter(2)

        plsc.subcore_barrier()
        pltpu.sync_copy(acc_sh.at[pl.ds(r0, RPT)],
                        out_hbm.at[cid, pl.ds(r0, RPT)])

        @pl.when(sid == 0)
        def _():
            pltpu.sync_copy(acc_sh.at[pl.ds(NSUB * RPT, RREM)],
                            out_hbm.at[cid, pl.ds(NSUB * RPT, RREM)])

    return sc_layer


_sc_layer = _make_sc_layer()


def _proj_body(x_ref, w_ref, b_ref, o_ref):
    acc = lax.dot_general(x_ref[...], w_ref[...], (((1,), (0,)), ((), ())),
                          preferred_element_type=jnp.float32)
    o_ref[...] = jnp.maximum(acc + b_ref[...], 0.0)


def _proj(xp, wap, ba2):
    return pl.pallas_call(
        _proj_body,
        out_shape=jax.ShapeDtypeStruct((N, D), jnp.float32),
    )(xp, wap, ba2)


_BLK = 2000


def _mlp_body(h_ref, agg_ref, w1_ref, b1_ref, w2_ref, b2_ref, o_ref):
    z = h_ref[...] + agg_ref[0] + agg_ref[1]
    z = jnp.maximum(
        lax.dot_general(z, w1_ref[...], (((1,), (0,)), ((), ())),
                        preferred_element_type=jnp.float32) + b1_ref[...], 0.0)
    z = lax.dot_general(z, w2_ref[...], (((1,), (0,)), ((), ())),
                        preferred_element_type=jnp.float32) + b2_ref[...]
    o_ref[...] = jnp.maximum(z, 0.0)


def _mlp(h, agg, w1, b12, w2, b22):
    return pl.pallas_call(
        _mlp_body,
        grid=(N // _BLK,),
        in_specs=[
            pl.BlockSpec((_BLK, D), lambda i: (i, 0)),
            pl.BlockSpec((2, _BLK, D), lambda i: (0, i, 0)),
            pl.BlockSpec((D, D), lambda i: (0, 0)),
            pl.BlockSpec((1, D), lambda i: (0, 0)),
            pl.BlockSpec((D, D), lambda i: (0, 0)),
            pl.BlockSpec((1, D), lambda i: (0, 0)),
        ],
        out_specs=pl.BlockSpec((_BLK, D), lambda i: (i, 0)),
        out_shape=jax.ShapeDtypeStruct((N, D), jnp.float32),
    )(h, agg, w1, b12, w2, b22)


def _pool_body(h_ref, b_ref, wh1_ref, bh1_ref, wh2_ref, bh2_ref, o_ref):
    bt = b_ref[...]
    io = lax.broadcasted_iota(jnp.int32, (G, N), 0)
    oht = (bt == io).astype(jnp.float32)
    g = lax.dot_general(oht, h_ref[...], (((1,), (0,)), ((), ())),
                        preferred_element_type=jnp.float32)
    q = jnp.maximum(
        lax.dot_general(g, wh1_ref[...], (((1,), (0,)), ((), ())),
                        preferred_element_type=jnp.float32) + bh1_ref[...], 0.0)
    o_ref[...] = lax.dot_general(q, wh2_ref[...], (((1,), (0,)), ((), ())),
                                 preferred_element_type=jnp.float32) + bh2_ref[...]


def _pool(h, batch2, wh1, bh12, wh2, bh22):
    return pl.pallas_call(
        _pool_body,
        out_shape=jax.ShapeDtypeStruct((G, 1), jnp.float32),
    )(h, batch2, wh1, bh12, wh2, bh22)


def kernel(x, edge_index, edge_attr, batch, Wa, ba, W1, b1, W2, b2,
           We1, be1, We2, be2, We3, be3, Wh1, bh1, Wh2, bh2):
    xp = jnp.pad(x, ((0, 0), (0, 16 - x.shape[1])))
    wap = jnp.pad(Wa, ((0, 16 - Wa.shape[0]), (0, 0)))
    h = _proj(xp, wap, ba.reshape(1, D))

    si = jnp.pad(edge_index[0], (0, EPAD - E))
    di = jnp.concatenate(
        [edge_index[1], N + (jnp.arange(EPAD - E, dtype=jnp.int32) % 8)])
    ea = jnp.pad(edge_attr.reshape(-1), (0, (EPAD - E) * ED))
    zeros = jnp.zeros((NP, D), jnp.float32)
    b12 = b1.reshape(1, D)
    b22 = b2.reshape(1, D)
    for We, be in ((We1, be1), (We2, be2), (We3, be3)):
        agg = _sc_layer(h, si, di, ea, We.reshape(-1), be, zeros)
        h = _mlp(h, agg, W1, b12, W2, b22)

    out = _pool(h, batch.reshape(1, N), Wh1, bh1.reshape(1, G),
                Wh2, bh2.reshape(1, 1))
    return out.reshape(-1)


# speedup vs baseline: 4.6281x; 1.3476x over previous
import functools

import jax
import jax.numpy as jnp
from jax import lax
from jax.experimental import pallas as pl
from jax.experimental.pallas import tpu as pltpu
from jax.experimental.pallas import tpu_sc as plsc

N = 10000
E = 320000
D = 128
ED = 4
G = 64

C = 72
NW = 32
NSLOT = 5
KPW = 140
NQ = KPW // NSLOT
EPAD = NW * KPW * C
NP = N + 8
NSUB = 16
RPT = 624
RREM = N - NSUB * RPT
ZREM = NP - NSUB * RPT


def _make_sc_layer():
    mesh = plsc.VectorSubcoreMesh(core_axis_name="c", subcore_axis_name="s")

    @functools.partial(
        pl.kernel,
        mesh=mesh,
        out_type=jax.ShapeDtypeStruct((2, N, D), jnp.float32),
        scratch_types=(
            [
                pltpu.VMEM((NSLOT, C, D), jnp.float32),
                pltpu.VMEM((NSLOT, C), jnp.int32),
                pltpu.VMEM((NSLOT, C), jnp.int32),
                pltpu.VMEM_SHARED((NP, D), jnp.float32),
            ]
            + [pltpu.SemaphoreType.DMA] * (3 * NSLOT)
        ),
    )
    def sc_layer(h_hbm, si_hbm, di_hbm, e_hbm, z_hbm, out_hbm,
                 rowsS, srcS, dstS, acc_sh, *sems):
        cid = lax.axis_index("c")
        sid = lax.axis_index("s")
        w = sid * 2 + cid
        rows = [rowsS.at[p] for p in range(NSLOT)]
        srcs = [srcS.at[p] for p in range(NSLOT)]
        dsts = [dstS.at[p] for p in range(NSLOT)]
        sgs = sems[0:NSLOT]
        sss = sems[NSLOT:2 * NSLOT]
        sis = sems[2 * NSLOT:3 * NSLOT]

        r0 = sid * RPT
        pltpu.sync_copy(z_hbm.at[pl.ds(r0, RPT)], acc_sh.at[pl.ds(r0, RPT)])

        @pl.when(sid == 0)
        def _():
            pltpu.sync_copy(z_hbm.at[pl.ds(NSUB * RPT, ZREM)],
                            acc_sh.at[pl.ds(NSUB * RPT, ZREM)])

        plsc.subcore_barrier()

        ebase = w * KPW * C

        def is_real(k):
            return ebase + k * C < E

        def issue_idx(k, p):
            base = ebase + k * C
            pltpu.async_copy(si_hbm.at[pl.ds(base, C)], srcs[p], sis[p])
            pltpu.async_copy(di_hbm.at[pl.ds(base, C)], dsts[p], sis[p])
            pltpu.async_copy(e_hbm.at[pl.ds(base, C)], rows[p], sis[p])

        def wait_idx(p):
            pltpu.make_async_copy(si_hbm.at[pl.ds(0, C)], srcs[p], sis[p]).wait()
            pltpu.make_async_copy(di_hbm.at[pl.ds(0, C)], dsts[p], sis[p]).wait()
            pltpu.make_async_copy(e_hbm.at[pl.ds(0, C)], rows[p], sis[p]).wait()

        def issue_gather(p):
            pltpu.async_copy(h_hbm.at[srcs[p]], rows[p], sgs[p], add=True)

        def wait_gather(p):
            pltpu.make_async_copy(h_hbm.at[pl.ds(0, C)], rows[p], sgs[p]).wait()

        def issue_scatter(p):
            pltpu.async_copy(rows[p], acc_sh.at[dsts[p]], sss[p], add=True)

        def wait_scatter(p):
            pltpu.make_async_copy(h_hbm.at[pl.ds(0, C)], rows[p], sss[p]).wait()

        def compute(p):
            rp = rows[p]

            def edge_body(j, carry):
                for g in range(8):
                    hv = rp[j, pl.ds(g * 16, 16)]
                    rp[j, pl.ds(g * 16, 16)] = jnp.maximum(hv, 0.0)
                return carry

            lax.fori_loop(0, C, edge_body, 0, unroll=2)

        for k in (0, 1, 2):
            @pl.when(is_real(k))
            def _(k=k):
                issue_idx(k, k)

        for k in (0, 1):
            @pl.when(is_real(k))
            def _(k=k):
                wait_idx(k)
                issue_gather(k)

        def macro(kq, carry):
            j0 = NSLOT * kq
            for p in range(NSLOT):
                j = j0 + p

                @pl.when(is_real(j))
                def _(p=p):
                    wait_gather(p)
                    compute(p)
                    issue_scatter(p)

                @pl.when((j >= 2) & is_real(j - 2))
                def _(p=p):
                    wait_scatter((p + 3) % NSLOT)

                @pl.when(is_real(j + 3) & (j + 3 < KPW))
                def _(j=j, p=p):
                    issue_idx(j + 3, (p + 3) % NSLOT)

                @pl.when(is_real(j + 2) & (j + 2 < KPW))
                def _(p=p):
                    wait_idx((p + 2) % NSLOT)
                    issue_gather((p + 2) % NSLOT)

            return carry

        lax.fori_loop(0, NQ, macro, 0, unroll=False)

        for k in (KPW - 2, KPW - 1):
            @pl.when(is_real(k))
            def _(k=k):
                wait_scatter(k % NSLOT)

        plsc.subcore_barrier()
        pltpu.sync_copy(acc_sh.at[pl.ds(r0, RPT)],
                        out_hbm.at[cid, pl.ds(r0, RPT)])

        @pl.when(sid == 0)
        def _():
            pltpu.sync_copy(acc_sh.at[pl.ds(NSUB * RPT, RREM)],
                            out_hbm.at[cid, pl.ds(NSUB * RPT, RREM)])

    return sc_layer


_sc_layer = _make_sc_layer()


def _proj_body(x_ref, w_ref, b_ref, o_ref):
    acc = lax.dot_general(x_ref[...], w_ref[...], (((1,), (0,)), ((), ())),
                          preferred_element_type=jnp.float32)
    o_ref[...] = jnp.maximum(acc + b_ref[...], 0.0)


def _proj(xp, wap, ba2):
    return pl.pallas_call(
        _proj_body,
        out_shape=jax.ShapeDtypeStruct((N, D), jnp.float32),
    )(xp, wap, ba2)


_EBLK = 3840


def _emb_body(a_ref, w_ref, b_ref, o_ref):
    o_ref[...] = lax.dot_general(a_ref[...], w_ref[...], (((1,), (0,)), ((), ())),
                                 preferred_element_type=jnp.float32) + b_ref[...]


def _emb(a, w, b2):
    return pl.pallas_call(
        _emb_body,
        grid=(EPAD // _EBLK,),
        in_specs=[
            pl.BlockSpec((_EBLK, ED), lambda i: (i, 0)),
            pl.BlockSpec((ED, D), lambda i: (0, 0)),
            pl.BlockSpec((1, D), lambda i: (0, 0)),
        ],
        out_specs=pl.BlockSpec((_EBLK, D), lambda i: (i, 0)),
        out_shape=jax.ShapeDtypeStruct((EPAD, D), jnp.float32),
    )(a, w, b2)


_BLK = 2000


def _mlp_body(h_ref, agg_ref, w1_ref, b1_ref, w2_ref, b2_ref, o_ref):
    z = h_ref[...] + agg_ref[0] + agg_ref[1]
    z = jnp.maximum(
        lax.dot_general(z, w1_ref[...], (((1,), (0,)), ((), ())),
                        preferred_element_type=jnp.float32) + b1_ref[...], 0.0)
    z = lax.dot_general(z, w2_ref[...], (((1,), (0,)), ((), ())),
                        preferred_element_type=jnp.float32) + b2_ref[...]
    o_ref[...] = jnp.maximum(z, 0.0)


def _mlp(h, agg, w1, b12, w2, b22):
    return pl.pallas_call(
        _mlp_body,
        grid=(N // _BLK,),
        in_specs=[
            pl.BlockSpec((_BLK, D), lambda i: (i, 0)),
            pl.BlockSpec((2, _BLK, D), lambda i: (0, i, 0)),
            pl.BlockSpec((D, D), lambda i: (0, 0)),
            pl.BlockSpec((1, D), lambda i: (0, 0)),
            pl.BlockSpec((D, D), lambda i: (0, 0)),
            pl.BlockSpec((1, D), lambda i: (0, 0)),
        ],
        out_specs=pl.BlockSpec((_BLK, D), lambda i: (i, 0)),
        out_shape=jax.ShapeDtypeStruct((N, D), jnp.float32),
    )(h, agg, w1, b12, w2, b22)


def _pool_body(h_ref, b_ref, wh1_ref, bh1_ref, wh2_ref, bh2_ref, o_ref):
    bt = b_ref[...]
    io = lax.broadcasted_iota(jnp.int32, (G, N), 0)
    oht = (bt == io).astype(jnp.float32)
    g = lax.dot_general(oht, h_ref[...], (((1,), (0,)), ((), ())),
                        preferred_element_type=jnp.float32)
    q = jnp.maximum(
        lax.dot_general(g, wh1_ref[...], (((1,), (0,)), ((), ())),
                        preferred_element_type=jnp.float32) + bh1_ref[...], 0.0)
    o_ref[...] = lax.dot_general(q, wh2_ref[...], (((1,), (0,)), ((), ())),
                                 preferred_element_type=jnp.float32) + bh2_ref[...]


def _pool(h, batch2, wh1, bh12, wh2, bh22):
    return pl.pallas_call(
        _pool_body,
        out_shape=jax.ShapeDtypeStruct((G, 1), jnp.float32),
    )(h, batch2, wh1, bh12, wh2, bh22)


def kernel(x, edge_index, edge_attr, batch, Wa, ba, W1, b1, W2, b2,
           We1, be1, We2, be2, We3, be3, Wh1, bh1, Wh2, bh2):
    xp = jnp.pad(x, ((0, 0), (0, 16 - x.shape[1])))
    wap = jnp.pad(Wa, ((0, 16 - Wa.shape[0]), (0, 0)))
    h = _proj(xp, wap, ba.reshape(1, D))

    si = jnp.pad(edge_index[0], (0, EPAD - E))
    di = jnp.concatenate(
        [edge_index[1], N + (jnp.arange(EPAD - E, dtype=jnp.int32) % 8)])
    eap = jnp.pad(edge_attr, ((0, EPAD - E), (0, 0)))
    zeros = jnp.zeros((NP, D), jnp.float32)
    b12 = b1.reshape(1, D)
    b22 = b2.reshape(1, D)
    for We, be in ((We1, be1), (We2, be2), (We3, be3)):
        e = _emb(eap, We, be.reshape(1, D))
        agg = _sc_layer(h, si, di, e, zeros)
        h = _mlp(h, agg, W1, b12, W2, b22)

    out = _pool(h, batch.reshape(1, N), Wh1, bh1.reshape(1, G),
                Wh2, bh2.reshape(1, 1))
    return out.reshape(-1)
